# baseline (speedup 1.0000x reference)
import functools

import jax
import jax.numpy as jnp
from jax import lax
from jax.experimental import pallas as pl
from jax.experimental.pallas import tpu as pltpu
from jax.experimental.pallas import tpu_sc as plsc


def _tree_sum(vals):
    while len(vals) > 1:
        pair = [vals[j] + vals[j + 1] for j in range(0, len(vals) - 1, 2)]
        if len(vals) % 2:
            pair.append(vals[-1])
        vals = pair
    return vals[0]


def _morph_kernel(B, D, AL, NC, NW, BW, CH, P1):
    NCHUNK = BW // CH
    NSTEP = NCHUNK * 3
    mesh = plsc.VectorSubcoreMesh(core_axis_name="c", subcore_axis_name="s")

    @functools.partial(
        pl.kernel,
        mesh=mesh,
        out_type=jax.ShapeDtypeStruct((B, D), jnp.float32),
        scratch_types=[
            pltpu.VMEM((BW,), jnp.int32),
            pltpu.VMEM((BW * AL,), jnp.int32),
            pltpu.VMEM((BW * AL,), jnp.int32),
            pltpu.VMEM((BW * AL,), jnp.int32),
            pltpu.VMEM((BW, D), jnp.float32),
            pltpu.VMEM((CH * AL, D), jnp.float32),
            pltpu.VMEM((CH * AL, D), jnp.float32),
            pltpu.VMEM((CH, D), jnp.float32),
            pltpu.VMEM_SHARED((P1, D), jnp.float32),
            pltpu.SemaphoreType.DMA,
            pltpu.SemaphoreType.DMA,
            pltpu.SemaphoreType.DMA,
        ],
        compiler_params=pltpu.CompilerParams(use_tc_tiling_on_sc=False),
    )
    def k(wt, pt, wih, fih, lih, pih, out, idx_w, idx_f, idx_l, idx_p,
          wrows, gbuf0, gbuf1, acc, pts, sem0, sem1, semw):
        cid = lax.axis_index("c")
        sid = lax.axis_index("s")
        wid = sid * NC + cid
        base = pl.multiple_of(wid * BW, BW)

        gb = (gbuf0, gbuf1)
        sems = (sem0, sem1)

        def start(step, tables):
            c, t = divmod(step, 3)
            idxr, tbl = tables[t]
            o = c * CH * AL
            return pltpu.async_copy(tbl.at[idxr.at[pl.ds(o, CH * AL)]],
                                    gb[step % 2], sems[step % 2])

        pltpu.sync_copy(fih.at[pl.ds(base * AL, BW * AL)], idx_f)
        tables = ((idx_f, wt), (idx_l, wt), (idx_p, pts))
        dma = {0: start(0, tables)}

        pltpu.sync_copy(lih.at[pl.ds(base * AL, BW * AL)], idx_l)
        pltpu.sync_copy(pih.at[pl.ds(base * AL, BW * AL)], idx_p)
        pltpu.sync_copy(wih.at[pl.ds(base, BW)], idx_w)
        wdma = pltpu.async_copy(wt.at[idx_w], wrows, semw)

        @pl.when(sid == 0)
        def _():
            pltpu.sync_copy(pt, pts)

        plsc.subcore_barrier()

        mscale = jnp.float32(0.25 / AL)
        wscale = jnp.float32(0.25)
        for step in range(NSTEP):
            c, t = divmod(step, 3)
            if step + 1 < NSTEP:
                dma[step + 1] = start(step + 1, tables)
            dma[step].wait()
            buf = gb[step % 2]
            if t == 2 and c == 0:
                wdma.wait()

            def word_body(i, _, t=t, c=c, buf=buf):
                r0 = i * AL
                h0 = _tree_sum([buf[r0 + r, pl.ds(0, 16)] for r in range(AL)])
                h1 = _tree_sum([buf[r0 + r, pl.ds(16, 16)] for r in range(AL)])
                if t == 0:
                    acc[i, pl.ds(0, 16)] = h0
                    acc[i, pl.ds(16, 16)] = h1
                elif t == 1:
                    acc[i, pl.ds(0, 16)] = acc[i, pl.ds(0, 16)] + h0
                    acc[i, pl.ds(16, 16)] = acc[i, pl.ds(16, 16)] + h1
                else:
                    w = c * CH + i
                    acc[i, pl.ds(0, 16)] = (
                        (acc[i, pl.ds(0, 16)] + h0) * mscale
                        + wrows[w, pl.ds(0, 16)] * wscale)
                    acc[i, pl.ds(16, 16)] = (
                        (acc[i, pl.ds(16, 16)] + h1) * mscale
                        + wrows[w, pl.ds(16, 16)] * wscale)
                return 0

            lax.fori_loop(0, CH, word_body, 0)

            if t == 2:
                pltpu.sync_copy(acc, out.at[pl.ds(base + c * CH, CH)])

    return k


def kernel(word_table, postag_table, word_idx, forms_idx, lemmas_idx,
           postags_idx):
    B = word_idx.shape[0]
    D = word_table.shape[1]
    AL = forms_idx.shape[1] * forms_idx.shape[2]
    P1 = postag_table.shape[0]
    info = plsc.get_sparse_core_info()
    NC, NS = info.num_cores, info.num_subcores
    NW = NC * NS
    BW = B // NW
    CH = 32

    wi = word_idx.astype(jnp.int32)
    fi = forms_idx.reshape(-1).astype(jnp.int32)
    li = lemmas_idx.reshape(-1).astype(jnp.int32)
    pi = postags_idx.reshape(-1).astype(jnp.int32)

    k = _morph_kernel(B, D, AL, NC, NW, BW, CH, P1)
    return k(word_table, postag_table, wi, fi, li, pi)


# speedup vs baseline: 1.0140x; 1.0140x over previous
import functools

import jax
import jax.numpy as jnp
from jax import lax
from jax.experimental import pallas as pl
from jax.experimental.pallas import tpu as pltpu
from jax.experimental.pallas import tpu_sc as plsc


def _tree_sum(vals):
    while len(vals) > 1:
        pair = [vals[j] + vals[j + 1] for j in range(0, len(vals) - 1, 2)]
        if len(vals) % 2:
            pair.append(vals[-1])
        vals = pair
    return vals[0]


def _morph_kernel(B, D, AL, NC, NW, BW, CH, P1):
    NCHUNK = BW // CH
    NSTEP = NCHUNK * 3
    mesh = plsc.VectorSubcoreMesh(core_axis_name="c", subcore_axis_name="s")

    @functools.partial(
        pl.kernel,
        mesh=mesh,
        out_type=jax.ShapeDtypeStruct((B, D), jnp.float32),
        scratch_types=[
            pltpu.VMEM((BW,), jnp.int32),
            pltpu.VMEM((BW * AL,), jnp.int32),
            pltpu.VMEM((BW * AL,), jnp.int32),
            pltpu.VMEM((BW * AL,), jnp.int32),
            pltpu.VMEM((BW, D), jnp.float32),
            pltpu.VMEM((CH * AL, D), jnp.float32),
            pltpu.VMEM((CH * AL, D), jnp.float32),
            pltpu.VMEM((CH, D), jnp.float32),
            pltpu.VMEM_SHARED((P1, D), jnp.float32),
            pltpu.SemaphoreType.DMA,
            pltpu.SemaphoreType.DMA,
            pltpu.SemaphoreType.DMA,
        ],
        compiler_params=pltpu.CompilerParams(use_tc_tiling_on_sc=False),
    )
    def k(wt, pt, wih, fih, lih, pih, out, idx_w, idx_f, idx_l, idx_p,
          wrows, gbuf0, gbuf1, acc, pts, sem0, sem1, semw):
        cid = lax.axis_index("c")
        sid = lax.axis_index("s")
        wid = sid * NC + cid
        base = pl.multiple_of(wid * BW, BW)

        gb = (gbuf0, gbuf1)
        sems = (sem0, sem1)

        def start(step, tables):
            c, t = divmod(step, 3)
            idxr, tbl = tables[t]
            o = c * CH * AL
            return pltpu.async_copy(tbl.at[idxr.at[pl.ds(o, CH * AL)]],
                                    gb[step % 2], sems[step % 2])

        pltpu.sync_copy(fih.at[pl.ds(base * AL, BW * AL)], idx_f)
        tables = ((idx_f, wt), (idx_l, wt), (idx_p, pts))
        dma = {0: start(0, tables)}

        pltpu.sync_copy(lih.at[pl.ds(base * AL, BW * AL)], idx_l)
        pltpu.sync_copy(pih.at[pl.ds(base * AL, BW * AL)], idx_p)
        pltpu.sync_copy(wih.at[pl.ds(base, BW)], idx_w)
        wdma = pltpu.async_copy(wt.at[idx_w], wrows, semw)

        @pl.when(sid == 0)
        def _():
            pltpu.sync_copy(pt, pts)

        plsc.subcore_barrier()

        mscale = jnp.float32(0.25 / AL)
        wscale = jnp.float32(0.25)
        for step in range(NSTEP):
            c, t = divmod(step, 3)
            if step + 1 < NSTEP:
                dma[step + 1] = start(step + 1, tables)
            dma[step].wait()
            buf = gb[step % 2]
            if t == 2 and c == 0:
                wdma.wait()

            def word_body(i, _, t=t, c=c, buf=buf):
                r0 = i * AL
                h0 = _tree_sum([buf[r0 + r, pl.ds(0, 16)] for r in range(AL)])
                h1 = _tree_sum([buf[r0 + r, pl.ds(16, 16)] for r in range(AL)])
                if t == 0:
                    acc[i, pl.ds(0, 16)] = h0
                    acc[i, pl.ds(16, 16)] = h1
                elif t == 1:
                    acc[i, pl.ds(0, 16)] = acc[i, pl.ds(0, 16)] + h0
                    acc[i, pl.ds(16, 16)] = acc[i, pl.ds(16, 16)] + h1
                else:
                    w = c * CH + i
                    acc[i, pl.ds(0, 16)] = (
                        (acc[i, pl.ds(0, 16)] + h0) * mscale
                        + wrows[w, pl.ds(0, 16)] * wscale)
                    acc[i, pl.ds(16, 16)] = (
                        (acc[i, pl.ds(16, 16)] + h1) * mscale
                        + wrows[w, pl.ds(16, 16)] * wscale)
                return 0

            lax.fori_loop(0, CH, word_body, 0)

            if t == 2:
                pltpu.sync_copy(acc, out.at[pl.ds(base + c * CH, CH)])

    return k


def kernel(word_table, postag_table, word_idx, forms_idx, lemmas_idx,
           postags_idx):
    B = word_idx.shape[0]
    D = word_table.shape[1]
    AL = forms_idx.shape[1] * forms_idx.shape[2]
    P1 = postag_table.shape[0]
    info = plsc.get_sparse_core_info()
    NC, NS = info.num_cores, info.num_subcores
    NW = NC * NS
    BW = B // NW
    CH = 64

    wi = word_idx.astype(jnp.int32)
    fi = forms_idx.reshape(-1).astype(jnp.int32)
    li = lemmas_idx.reshape(-1).astype(jnp.int32)
    pi = postags_idx.reshape(-1).astype(jnp.int32)

    k = _morph_kernel(B, D, AL, NC, NW, BW, CH, P1)
    return k(word_table, postag_table, wi, fi, li, pi)
